# baseline (speedup 1.0000x reference)
import jax
import jax.numpy as jnp
from jax.experimental import pallas as pl
from jax.experimental.pallas import tpu as pltpu

NROW, NCOL = 160, 125
NC = 256
K = 200
CONF_THRESH_ = 0.01
NMS_THRESH_ = 0.45
NEG_INF = float("-inf")


def _ssd_body(loc_ref, pri_ref, cs_ref, rows_ref,
              x1_ref, y1_ref, x2_ref, y2_ref, sc_ref):
    lx, ly, lw, lh = loc_ref[0], loc_ref[1], loc_ref[2], loc_ref[3]
    px, py, pw, ph = pri_ref[0], pri_ref[1], pri_ref[2], pri_ref[3]
    s = cs_ref[...]

    w = pw * jnp.exp(lw * 0.2)
    h = ph * jnp.exp(lh * 0.2)
    x1 = (px + (lx * 0.1) * pw) - w / 2.0
    y1 = (py + (ly * 0.1) * ph) - h / 2.0
    x2 = w + x1
    y2 = h + y1
    x1_ref[...] = x1
    y1_ref[...] = y1
    x2_ref[...] = x2
    y2_ref[...] = y2

    masked = jnp.where(s > CONF_THRESH_, s, NEG_INF)
    sc_ref[...] = masked
    rowmax = jnp.max(masked, axis=1).reshape(1, NROW)

    lane_r = jax.lax.broadcasted_iota(jnp.int32, (1, NROW), 1)
    lane_c = jax.lax.broadcasted_iota(jnp.int32, (1, NCOL), 1)
    lane_k = jax.lax.broadcasted_iota(jnp.int32, (1, NC), 1)
    zrow = jnp.zeros((1, NC), jnp.float32)

    def body(t, carry):
        rmax, cnt, ks, kx1, ky1, kx2, ky2, karea, kval = carry
        m = jnp.max(rmax)
        j = jnp.max(jnp.where(rmax == m, lane_r, -1))
        slab = sc_ref[pl.ds(j, 1), :]
        li = jnp.max(jnp.where(slab == m, lane_c, -1))
        emask = lane_c == li
        ex1 = jnp.sum(jnp.where(emask, x1_ref[pl.ds(j, 1), :], 0.0))
        ey1 = jnp.sum(jnp.where(emask, y1_ref[pl.ds(j, 1), :], 0.0))
        ex2 = jnp.sum(jnp.where(emask, x2_ref[pl.ds(j, 1), :], 0.0))
        ey2 = jnp.sum(jnp.where(emask, y2_ref[pl.ds(j, 1), :], 0.0))
        area_t = (ex2 - ex1) * (ey2 - ey1)
        xx1 = jnp.maximum(kx1, ex1)
        yy1 = jnp.maximum(ky1, ey1)
        xx2 = jnp.minimum(kx2, ex2)
        yy2 = jnp.minimum(ky2, ey2)
        wv = jnp.maximum(xx2 - xx1, 0.0)
        hv = jnp.maximum(yy2 - yy1, 0.0)
        inter = wv * hv
        union = (area_t - inter) + karea
        iou = inter / union
        over = (kval > 0) & jnp.logical_not(iou <= NMS_THRESH_)
        suppr = jnp.max(jnp.where(over, 1, 0)) > 0
        keep = (m > CONF_THRESH_) & jnp.logical_not(suppr)
        selc = (lane_k == cnt) & keep
        ks = jnp.where(selc, m, ks)
        kx1 = jnp.where(selc, ex1, kx1)
        ky1 = jnp.where(selc, ey1, ky1)
        kx2 = jnp.where(selc, ex2, kx2)
        ky2 = jnp.where(selc, ey2, ky2)
        karea = jnp.where(selc, area_t, karea)
        kval = jnp.where(selc, 1, kval)
        cnt = jnp.where(keep, cnt + 1, cnt)
        slab2 = jnp.where(emask, NEG_INF, slab)
        sc_ref[pl.ds(j, 1), :] = slab2
        rmax = jnp.where(lane_r == j, jnp.max(slab2), rmax)
        return rmax, cnt, ks, kx1, ky1, kx2, ky2, karea, kval

    carry0 = (rowmax, jnp.int32(0), zrow, zrow, zrow, zrow, zrow, zrow,
              jnp.zeros((1, NC), jnp.int32))
    _, _, ks, kx1, ky1, kx2, ky2, _, _ = jax.lax.fori_loop(0, K, body, carry0)

    pack = jnp.concatenate(
        [ks, kx1, ky1, kx2, ky2, jnp.zeros((3, NC), jnp.float32)], axis=0)
    rows_ref[...] = pack.T


def kernel(loc_data, conf_data, prior_data):
    loc_t = loc_data[0].T.reshape(4, NROW, NCOL)
    pri_t = prior_data.T.reshape(4, NROW, NCOL)
    cs = conf_data[0, :, 1].reshape(NROW, NCOL)

    rows = pl.pallas_call(
        _ssd_body,
        out_shape=jax.ShapeDtypeStruct((NC, 8), jnp.float32),
        scratch_shapes=[
            pltpu.VMEM((NROW, NCOL), jnp.float32),
            pltpu.VMEM((NROW, NCOL), jnp.float32),
            pltpu.VMEM((NROW, NCOL), jnp.float32),
            pltpu.VMEM((NROW, NCOL), jnp.float32),
            pltpu.VMEM((NROW, NCOL), jnp.float32),
        ],
    )(loc_t, pri_t, cs)

    out = jnp.zeros((1, 2, K, 5), jnp.float32)
    return out.at[0, 1].set(rows[:K, :5])


# speedup vs baseline: 1.1120x; 1.1120x over previous
import jax
import jax.numpy as jnp
from jax.experimental import pallas as pl
from jax.experimental.pallas import tpu as pltpu

NROW, NCOL = 160, 125
NC = 256
K = 200
CONF_THRESH_ = 0.01
NMS_THRESH_ = 0.45
NEG_INF = float("-inf")


def _ssd_body(loc_ref, pri_ref, cs_ref, rows_ref,
              x1_ref, y1_ref, x2_ref, y2_ref, sc_ref, cand_ref, iou_ref):
    lx, ly, lw, lh = loc_ref[0], loc_ref[1], loc_ref[2], loc_ref[3]
    px, py, pw, ph = pri_ref[0], pri_ref[1], pri_ref[2], pri_ref[3]
    s = cs_ref[...]

    w = pw * jnp.exp(lw * 0.2)
    h = ph * jnp.exp(lh * 0.2)
    x1 = (px + (lx * 0.1) * pw) - w / 2.0
    y1 = (py + (ly * 0.1) * ph) - h / 2.0
    x2 = w + x1
    y2 = h + y1
    x1_ref[...] = x1
    y1_ref[...] = y1
    x2_ref[...] = x2
    y2_ref[...] = y2

    masked = jnp.where(s > CONF_THRESH_, s, NEG_INF)
    sc_ref[...] = masked
    rowmax = jnp.max(masked, axis=1).reshape(1, NROW)
    rows_ref[...] = jnp.zeros((NC, 8), jnp.float32)

    lane_r = jax.lax.broadcasted_iota(jnp.int32, (1, NROW), 1)
    lane_c = jax.lax.broadcasted_iota(jnp.int32, (1, NCOL), 1)
    lane256 = jax.lax.broadcasted_iota(jnp.int32, (1, NC), 1)
    zrow = jnp.zeros((1, NC), jnp.float32)

    def sel_body(t, carry):
        rmax, vs, vx1, vy1, vx2, vy2, vval = carry
        m = jnp.max(rmax)
        j = jnp.max(jnp.where(rmax == m, lane_r, -1))
        slab = sc_ref[pl.ds(j, 1), :]
        li = jnp.max(jnp.where(slab == m, lane_c, -1))
        emask = lane_c == li
        ex1 = jnp.sum(jnp.where(emask, x1_ref[pl.ds(j, 1), :], 0.0))
        ey1 = jnp.sum(jnp.where(emask, y1_ref[pl.ds(j, 1), :], 0.0))
        ex2 = jnp.sum(jnp.where(emask, x2_ref[pl.ds(j, 1), :], 0.0))
        ey2 = jnp.sum(jnp.where(emask, y2_ref[pl.ds(j, 1), :], 0.0))
        selt = lane256 == t
        vs = jnp.where(selt, m, vs)
        vx1 = jnp.where(selt, ex1, vx1)
        vy1 = jnp.where(selt, ey1, vy1)
        vx2 = jnp.where(selt, ex2, vx2)
        vy2 = jnp.where(selt, ey2, vy2)
        vval = jnp.where(selt & (m > CONF_THRESH_), 1, vval)
        slab2 = jnp.where(emask, NEG_INF, slab)
        sc_ref[pl.ds(j, 1), :] = slab2
        rmax = jnp.where(lane_r == j, jnp.max(slab2), rmax)
        return rmax, vs, vx1, vy1, vx2, vy2, vval

    carry0 = (rowmax, zrow, zrow, zrow, zrow, zrow,
              jnp.zeros((1, NC), jnp.int32))
    _, vs, vx1, vy1, vx2, vy2, vval = jax.lax.fori_loop(0, K, sel_body, carry0)

    pack = jnp.concatenate(
        [vs, vx1, vy1, vx2, vy2, jnp.zeros((3, NC), jnp.float32)], axis=0)
    cand_ref[...] = pack.T

    area_r = (vx2 - vx1) * (vy2 - vy1)
    x1c = cand_ref[:, 1:2]
    y1c = cand_ref[:, 2:3]
    x2c = cand_ref[:, 3:4]
    y2c = cand_ref[:, 4:5]
    area_c = (x2c - x1c) * (y2c - y1c)
    xx1 = jnp.maximum(vx1, x1c)
    yy1 = jnp.maximum(vy1, y1c)
    xx2 = jnp.minimum(vx2, x2c)
    yy2 = jnp.minimum(vy2, y2c)
    wv = jnp.maximum(xx2 - xx1, 0.0)
    hv = jnp.maximum(yy2 - yy1, 0.0)
    inter = wv * hv
    union = (area_r - inter) + area_c
    iou_ref[...] = inter / union

    p0 = jnp.min(jnp.where(vval > 0, lane256, NC))

    def nms_cond(st):
        t, p, _ = st
        return (t < K) & (p < NC)

    def nms_body(st):
        t, p, active = st
        rows_ref[pl.ds(t, 1), :] = cand_ref[pl.ds(p, 1), :]
        row_iou = iou_ref[pl.ds(p, 1), :]
        keep = (row_iou <= NMS_THRESH_) & (lane256 != p)
        active2 = jnp.where(keep, active, 0)
        p2 = jnp.min(jnp.where(active2 > 0, lane256, NC))
        return t + 1, p2, active2

    jax.lax.while_loop(nms_cond, nms_body, (0, p0, vval))


def kernel(loc_data, conf_data, prior_data):
    loc_t = loc_data[0].T.reshape(4, NROW, NCOL)
    pri_t = prior_data.T.reshape(4, NROW, NCOL)
    cs = conf_data[0, :, 1].reshape(NROW, NCOL)

    rows = pl.pallas_call(
        _ssd_body,
        out_shape=jax.ShapeDtypeStruct((NC, 8), jnp.float32),
        scratch_shapes=[
            pltpu.VMEM((NROW, NCOL), jnp.float32),
            pltpu.VMEM((NROW, NCOL), jnp.float32),
            pltpu.VMEM((NROW, NCOL), jnp.float32),
            pltpu.VMEM((NROW, NCOL), jnp.float32),
            pltpu.VMEM((NROW, NCOL), jnp.float32),
            pltpu.VMEM((NC, 8), jnp.float32),
            pltpu.VMEM((NC, NC), jnp.float32),
        ],
    )(loc_t, pri_t, cs)

    out = jnp.zeros((1, 2, K, 5), jnp.float32)
    return out.at[0, 1].set(rows[:K, :5])
